# Initial kernel scaffold; baseline (speedup 1.0000x reference)
import jax
import jax.numpy as jnp
from jax.experimental import pallas as pl


def kernel(x, edge_index, W1, b1, W2, b2, W3, b3):
    raise NotImplementedError("write your pallas kernel here")


# speedup vs baseline: 11.8286x; 11.8286x over previous
import functools

import jax
import jax.numpy as jnp
from jax import lax
from jax.experimental import pallas as pl
from jax.experimental.pallas import tpu as pltpu
from jax.experimental.pallas import tpu_sc as plsc

N = 10000
E = 320000
D_IN = 128
D_H = 128
D_OUT = 64

NC = 2
NS = 16
NW = NC * NS
EPW = E // NW
B = 80
NCH = EPW // B
NPAD = 10240
SPT = NPAD // NS
RPT = NPAD // NS

_mesh = plsc.VectorSubcoreMesh(core_axis_name="c", subcore_axis_name="s")


def _zero_fill(ref, nelem):
    z16 = jnp.zeros((16,), jnp.float32)

    @pl.loop(0, nelem // 16)
    def _(i):
        ref[pl.ds(i * 16, 16)] = z16


def _deg_body(srcr, dstr, degs_out, dego_sh, degi_sh, stag_s, stag_d,
              ones_v, zero_v):
    cid = lax.axis_index("c")
    sid = lax.axis_index("s")
    w = cid * NS + sid
    base = sid * SPT

    o16 = jnp.ones((16,), jnp.float32)

    @pl.loop(0, B // 16)
    def _(i):
        ones_v[pl.ds(i * 16, 16)] = o16

    _zero_fill(zero_v, SPT)
    pltpu.sync_copy(zero_v, dego_sh.at[pl.ds(base, SPT)])
    pltpu.sync_copy(zero_v, degi_sh.at[pl.ds(base, SPT)])
    plsc.subcore_barrier()

    pltpu.sync_copy(srcr.at[w], stag_s)
    pltpu.sync_copy(dstr.at[w], stag_d)

    @pl.loop(0, NCH)
    def _(j):
        pltpu.sync_copy(ones_v, dego_sh.at[stag_s.at[j]], add=True)
        pltpu.sync_copy(ones_v, degi_sh.at[stag_d.at[j]], add=True)

    plsc.subcore_barrier()
    pltpu.sync_copy(dego_sh.at[pl.ds(base, SPT)],
                    degs_out.at[cid, 0, pl.ds(base, SPT)])
    pltpu.sync_copy(degi_sh.at[pl.ds(base, SPT)],
                    degs_out.at[cid, 1, pl.ds(base, SPT)])


_deg_kernel = pl.kernel(
    _deg_body,
    out_type=jax.ShapeDtypeStruct((NC, 2, NPAD), jnp.float32),
    mesh=_mesh,
    scratch_types=[
        pltpu.VMEM_SHARED((NPAD,), jnp.float32),
        pltpu.VMEM_SHARED((NPAD,), jnp.float32),
        pltpu.VMEM((NCH, B), jnp.int32),
        pltpu.VMEM((NCH, B), jnp.int32),
        pltpu.VMEM((B,), jnp.float32),
        pltpu.VMEM((SPT,), jnp.float32),
    ],
)


def _c_body(dstr, srcr, normd, c_out, c_sh, stag_s, stag_d,
            vals_v, zero_v, sem):
    cid = lax.axis_index("c")
    sid = lax.axis_index("s")
    w = cid * NS + sid
    base = sid * SPT

    _zero_fill(zero_v, SPT)
    pltpu.sync_copy(zero_v, c_sh.at[pl.ds(base, SPT)])
    plsc.subcore_barrier()

    pltpu.sync_copy(srcr.at[w], stag_s)
    pltpu.sync_copy(dstr.at[w], stag_d)

    @pl.loop(0, NCH)
    def _(j):
        pltpu.async_copy(normd.at[stag_d.at[j]], vals_v, sem).wait()
        pltpu.sync_copy(vals_v, c_sh.at[stag_s.at[j]], add=True)

    plsc.subcore_barrier()
    pltpu.sync_copy(c_sh.at[pl.ds(base, SPT)],
                    c_out.at[cid, pl.ds(base, SPT)])


_c_kernel = pl.kernel(
    _c_body,
    out_type=jax.ShapeDtypeStruct((NC, NPAD), jnp.float32),
    mesh=_mesh,
    scratch_types=[
        pltpu.VMEM_SHARED((NPAD,), jnp.float32),
        pltpu.VMEM((NCH, B), jnp.int32),
        pltpu.VMEM((NCH, B), jnp.int32),
        pltpu.VMEM((B,), jnp.float32),
        pltpu.VMEM((SPT,), jnp.float32),
        pltpu.SemaphoreType.DMA,
    ],
)


def _prop_body(g, srcr, dstr, zrows, ma, mb, acc_sh, stag_s, stag_d,
               rows_v, sem):
    cid = lax.axis_index("c")
    sid = lax.axis_index("s")
    w = cid * NS + sid
    rbase = sid * RPT

    @pl.when(cid == 0)
    def _():
        pltpu.sync_copy(g.at[pl.ds(rbase, RPT)], acc_sh.at[pl.ds(rbase, RPT)])

    @pl.when(cid == 1)
    def _():
        pltpu.sync_copy(zrows, acc_sh.at[pl.ds(rbase, RPT)])

    pltpu.sync_copy(srcr.at[w], stag_s)
    pltpu.sync_copy(dstr.at[w], stag_d)
    plsc.subcore_barrier()

    @pl.loop(0, NCH)
    def _(j):
        pltpu.async_copy(g.at[stag_s.at[j]], rows_v, sem).wait()
        pltpu.sync_copy(rows_v, acc_sh.at[stag_d.at[j]], add=True)

    plsc.subcore_barrier()

    @pl.when(cid == 0)
    def _():
        pltpu.sync_copy(acc_sh.at[pl.ds(rbase, RPT)], ma.at[pl.ds(rbase, RPT)])

    @pl.when(cid == 1)
    def _():
        pltpu.sync_copy(acc_sh.at[pl.ds(rbase, RPT)], mb.at[pl.ds(rbase, RPT)])


_prop_kernel = pl.kernel(
    _prop_body,
    out_type=(
        jax.ShapeDtypeStruct((NPAD, D_H), jnp.float32),
        jax.ShapeDtypeStruct((NPAD, D_H), jnp.float32),
    ),
    mesh=_mesh,
    scratch_types=[
        pltpu.VMEM_SHARED((NPAD, D_H), jnp.float32),
        pltpu.VMEM((NCH, B), jnp.int32),
        pltpu.VMEM((NCH, B), jnp.int32),
        pltpu.VMEM((B, D_H), jnp.float32),
        pltpu.SemaphoreType.DMA,
    ],
)


def _tc1_body(x_ref, ns_ref, w_ref, o_ref):
    o_ref[...] = jnp.dot(x_ref[...] * ns_ref[...], w_ref[...],
                         preferred_element_type=jnp.float32)


def _tc2_body(ma_ref, mb_ref, nd_ref, ns_ref, b_ref, w_ref, o_ref):
    h = jnp.maximum((ma_ref[...] + mb_ref[...]) * nd_ref[...] + b_ref[...],
                    0.0)
    o_ref[...] = jnp.dot(h * ns_ref[...], w_ref[...],
                         preferred_element_type=jnp.float32)


def _tc3_body(ma_ref, mb_ref, nd_ref, b2_ref, wv_ref, w3_ref, b3_ref, o_ref):
    h = jnp.maximum((ma_ref[...] + mb_ref[...]) * nd_ref[...] + b2_ref[...],
                    0.0)
    u = jnp.sum(h * wv_ref[...], axis=0, keepdims=True)
    o_ref[...] = jnp.dot(u, w3_ref[...],
                         preferred_element_type=jnp.float32) + b3_ref[...]


_tc1 = pl.pallas_call(
    _tc1_body, out_shape=jax.ShapeDtypeStruct((NPAD, D_H), jnp.float32))
_tc2 = pl.pallas_call(
    _tc2_body, out_shape=jax.ShapeDtypeStruct((NPAD, D_H), jnp.float32))
_tc3 = pl.pallas_call(
    _tc3_body, out_shape=jax.ShapeDtypeStruct((1, D_OUT), jnp.float32))


def kernel(x, edge_index, W1, b1, W2, b2, W3, b3):
    src = edge_index[0]
    dst = edge_index[1]
    srcr = src.reshape(NW, NCH, B)
    dstr = dst.reshape(NW, NCH, B)
    zrows = jnp.zeros((RPT, D_H), jnp.float32)
    x_pad = jnp.pad(x, ((0, NPAD - N), (0, 0)))

    degs = _deg_kernel(srcr, dstr)
    pad0 = jnp.zeros((NPAD - N,), jnp.float32)
    norm_src = jnp.concatenate(
        [lax.rsqrt(degs[0, 0, :N] + degs[1, 0, :N] + 1.0), pad0])
    norm_dst = jnp.concatenate(
        [lax.rsqrt(degs[0, 1, :N] + degs[1, 1, :N] + 1.0), pad0])

    cparts = _c_kernel(dstr, srcr, norm_dst)

    g1 = _tc1(x_pad, norm_src[:, None], W1)
    m1a, m1b = _prop_kernel(g1, srcr, dstr, zrows)

    g2 = _tc2(m1a, m1b, norm_dst[:, None], norm_src[:, None],
              b1.reshape(1, D_H), W2)
    m2a, m2b = _prop_kernel(g2, srcr, dstr, zrows)

    c = cparts[0] + cparts[1] + norm_dst
    wvec = (c * norm_src * (1.0 / N))[:, None]
    out = _tc3(m2a, m2b, norm_dst[:, None], b2.reshape(1, D_H), wvec,
               W3, b3.reshape(1, D_OUT))
    return out.reshape(D_OUT)


# speedup vs baseline: 20.1986x; 1.7076x over previous
import jax
import jax.numpy as jnp
from jax import lax
from jax.experimental import pallas as pl
from jax.experimental.pallas import tpu as pltpu
from jax.experimental.pallas import tpu_sc as plsc

N = 10000
E = 320000
D_IN = 128
D_H = 128
D_OUT = 64

NC = 2
NS = 16
NW = NC * NS
EPW = E // NW
B = 80
NCH = EPW // B
NB = 5
CPB = NCH // NB
NPAD = 10240
SPT = NPAD // NS
RPT = NPAD // NS

_mesh = plsc.VectorSubcoreMesh(core_axis_name="c", subcore_axis_name="s")


def _zero_fill(ref, nelem):
    z16 = jnp.zeros((16,), jnp.float32)

    @pl.loop(0, nelem // 16)
    def _(i):
        ref[pl.ds(i * 16, 16)] = z16


def _deg_body(srcr, dstr, degs_out, dego_sh, degi_sh, stag_s, stag_d,
              ones_v, zero_v):
    cid = lax.axis_index("c")
    sid = lax.axis_index("s")
    w = cid * NS + sid
    base = sid * SPT

    o16 = jnp.ones((16,), jnp.float32)

    @pl.loop(0, B // 16)
    def _(i):
        ones_v[pl.ds(i * 16, 16)] = o16

    _zero_fill(zero_v, SPT)
    pltpu.sync_copy(zero_v, dego_sh.at[pl.ds(base, SPT)])
    pltpu.sync_copy(zero_v, degi_sh.at[pl.ds(base, SPT)])
    plsc.subcore_barrier()

    pltpu.sync_copy(srcr.at[w], stag_s)
    pltpu.sync_copy(dstr.at[w], stag_d)

    @pl.loop(0, NB)
    def _(b):
        @pl.loop(0, CPB)
        def _(j):
            pltpu.sync_copy(ones_v, dego_sh.at[stag_s.at[b, j]], add=True)
            pltpu.sync_copy(ones_v, degi_sh.at[stag_d.at[b, j]], add=True)

    plsc.subcore_barrier()
    pltpu.sync_copy(dego_sh.at[pl.ds(base, SPT)],
                    degs_out.at[cid, 0, pl.ds(base, SPT)])
    pltpu.sync_copy(degi_sh.at[pl.ds(base, SPT)],
                    degs_out.at[cid, 1, pl.ds(base, SPT)])


_deg_kernel = pl.kernel(
    _deg_body,
    out_type=jax.ShapeDtypeStruct((NC, 2, NPAD), jnp.float32),
    mesh=_mesh,
    scratch_types=[
        pltpu.VMEM_SHARED((NPAD,), jnp.float32),
        pltpu.VMEM_SHARED((NPAD,), jnp.float32),
        pltpu.VMEM((NB, CPB, B), jnp.int32),
        pltpu.VMEM((NB, CPB, B), jnp.int32),
        pltpu.VMEM((B,), jnp.float32),
        pltpu.VMEM((SPT,), jnp.float32),
    ],
)


def _prop_c_body(g, srcr, dstr, zrows, normd, ma, mb, c_out,
                 acc_sh, c_sh, stag_s, stag_d, rows0, rows1, vals0, vals1,
                 zero_v, sem0, sem1, sem2, sem3):
    cid = lax.axis_index("c")
    sid = lax.axis_index("s")
    w = cid * NS + sid
    rbase = sid * RPT
    base = sid * SPT

    _zero_fill(zero_v, SPT)
    pltpu.sync_copy(zero_v, c_sh.at[pl.ds(base, SPT)])

    @pl.when(cid == 0)
    def _():
        pltpu.sync_copy(g.at[pl.ds(rbase, RPT)], acc_sh.at[pl.ds(rbase, RPT)])

    @pl.when(cid == 1)
    def _():
        pltpu.sync_copy(zrows, acc_sh.at[pl.ds(rbase, RPT)])

    plsc.subcore_barrier()

    def rgather(j, buf, sm):
        pltpu.async_copy(g.at[stag_s.at[j]], buf, sm)

    def rdrain(buf, sm):
        pltpu.make_async_copy(g.at[stag_s.at[0]], buf, sm).wait()

    def cgather(j, buf, sm):
        pltpu.async_copy(normd.at[stag_d.at[j]], buf, sm)

    def cdrain(buf, sm):
        pltpu.make_async_copy(normd.at[stag_d.at[0]], buf, sm).wait()

    @pl.loop(0, NB)
    def _(b):
        pltpu.sync_copy(srcr.at[w, b], stag_s)
        pltpu.sync_copy(dstr.at[w, b], stag_d)
        rgather(0, rows0, sem0)
        cgather(0, vals0, sem2)

        @pl.loop(0, (CPB - 1) // 2)
        def _(i):
            j = i * 2
            rgather(j + 1, rows1, sem1)
            cgather(j + 1, vals1, sem3)
            rdrain(rows0, sem0)
            pltpu.sync_copy(rows0, acc_sh.at[stag_d.at[j]], add=True)
            cdrain(vals0, sem2)
            pltpu.sync_copy(vals0, c_sh.at[stag_s.at[j]], add=True)
            rgather(j + 2, rows0, sem0)
            cgather(j + 2, vals0, sem2)
            rdrain(rows1, sem1)
            pltpu.sync_copy(rows1, acc_sh.at[stag_d.at[j + 1]], add=True)
            cdrain(vals1, sem3)
            pltpu.sync_copy(vals1, c_sh.at[stag_s.at[j + 1]], add=True)

        rdrain(rows0, sem0)
        pltpu.sync_copy(rows0, acc_sh.at[stag_d.at[CPB - 1]], add=True)
        cdrain(vals0, sem2)
        pltpu.sync_copy(vals0, c_sh.at[stag_s.at[CPB - 1]], add=True)

    plsc.subcore_barrier()

    @pl.when(cid == 0)
    def _():
        pltpu.sync_copy(acc_sh.at[pl.ds(rbase, RPT)], ma.at[pl.ds(rbase, RPT)])

    @pl.when(cid == 1)
    def _():
        pltpu.sync_copy(acc_sh.at[pl.ds(rbase, RPT)], mb.at[pl.ds(rbase, RPT)])

    pltpu.sync_copy(c_sh.at[pl.ds(base, SPT)],
                    c_out.at[cid, pl.ds(base, SPT)])


_prop_c_kernel = pl.kernel(
    _prop_c_body,
    out_type=(
        jax.ShapeDtypeStruct((NPAD, D_H), jnp.float32),
        jax.ShapeDtypeStruct((NPAD, D_H), jnp.float32),
        jax.ShapeDtypeStruct((NC, NPAD), jnp.float32),
    ),
    mesh=_mesh,
    scratch_types=[
        pltpu.VMEM_SHARED((NPAD, D_H), jnp.float32),
        pltpu.VMEM_SHARED((NPAD,), jnp.float32),
        pltpu.VMEM((CPB, B), jnp.int32),
        pltpu.VMEM((CPB, B), jnp.int32),
        pltpu.VMEM((B, D_H), jnp.float32),
        pltpu.VMEM((B, D_H), jnp.float32),
        pltpu.VMEM((B,), jnp.float32),
        pltpu.VMEM((B,), jnp.float32),
        pltpu.VMEM((SPT,), jnp.float32),
        pltpu.SemaphoreType.DMA,
        pltpu.SemaphoreType.DMA,
        pltpu.SemaphoreType.DMA,
        pltpu.SemaphoreType.DMA,
    ],
)


def _prop_body(g, srcr, dstr, zrows, ma, mb, acc_sh, stag_s, stag_d,
               rows0, rows1, sem0, sem1):
    cid = lax.axis_index("c")
    sid = lax.axis_index("s")
    w = cid * NS + sid
    rbase = sid * RPT

    @pl.when(cid == 0)
    def _():
        pltpu.sync_copy(g.at[pl.ds(rbase, RPT)], acc_sh.at[pl.ds(rbase, RPT)])

    @pl.when(cid == 1)
    def _():
        pltpu.sync_copy(zrows, acc_sh.at[pl.ds(rbase, RPT)])

    plsc.subcore_barrier()

    def gather(j, buf, sm):
        pltpu.async_copy(g.at[stag_s.at[j]], buf, sm)

    def drain(buf, sm):
        pltpu.make_async_copy(g.at[stag_s.at[0]], buf, sm).wait()

    @pl.loop(0, NB)
    def _(b):
        pltpu.sync_copy(srcr.at[w, b], stag_s)
        pltpu.sync_copy(dstr.at[w, b], stag_d)
        gather(0, rows0, sem0)

        @pl.loop(0, (CPB - 1) // 2)
        def _(i):
            j = i * 2
            gather(j + 1, rows1, sem1)
            drain(rows0, sem0)
            pltpu.sync_copy(rows0, acc_sh.at[stag_d.at[j]], add=True)
            gather(j + 2, rows0, sem0)
            drain(rows1, sem1)
            pltpu.sync_copy(rows1, acc_sh.at[stag_d.at[j + 1]], add=True)

        drain(rows0, sem0)
        pltpu.sync_copy(rows0, acc_sh.at[stag_d.at[CPB - 1]], add=True)

    plsc.subcore_barrier()

    @pl.when(cid == 0)
    def _():
        pltpu.sync_copy(acc_sh.at[pl.ds(rbase, RPT)], ma.at[pl.ds(rbase, RPT)])

    @pl.when(cid == 1)
    def _():
        pltpu.sync_copy(acc_sh.at[pl.ds(rbase, RPT)], mb.at[pl.ds(rbase, RPT)])


_prop_kernel = pl.kernel(
    _prop_body,
    out_type=(
        jax.ShapeDtypeStruct((NPAD, D_H), jnp.float32),
        jax.ShapeDtypeStruct((NPAD, D_H), jnp.float32),
    ),
    mesh=_mesh,
    scratch_types=[
        pltpu.VMEM_SHARED((NPAD, D_H), jnp.float32),
        pltpu.VMEM((CPB, B), jnp.int32),
        pltpu.VMEM((CPB, B), jnp.int32),
        pltpu.VMEM((B, D_H), jnp.float32),
        pltpu.VMEM((B, D_H), jnp.float32),
        pltpu.SemaphoreType.DMA,
        pltpu.SemaphoreType.DMA,
    ],
)


def _tc1_body(x_ref, ns_ref, w_ref, o_ref):
    o_ref[...] = jnp.dot(x_ref[...] * ns_ref[...], w_ref[...],
                         preferred_element_type=jnp.float32)


def _tc2_body(ma_ref, mb_ref, nd_ref, ns_ref, b_ref, w_ref, o_ref):
    h = jnp.maximum((ma_ref[...] + mb_ref[...]) * nd_ref[...] + b_ref[...],
                    0.0)
    o_ref[...] = jnp.dot(h * ns_ref[...], w_ref[...],
                         preferred_element_type=jnp.float32)


def _tc3_body(ma_ref, mb_ref, nd_ref, b2_ref, wv_ref, w3_ref, b3_ref, o_ref):
    h = jnp.maximum((ma_ref[...] + mb_ref[...]) * nd_ref[...] + b2_ref[...],
                    0.0)
    u = jnp.sum(h * wv_ref[...], axis=0, keepdims=True)
    o_ref[...] = jnp.dot(u, w3_ref[...],
                         preferred_element_type=jnp.float32) + b3_ref[...]


_tc1 = pl.pallas_call(
    _tc1_body, out_shape=jax.ShapeDtypeStruct((NPAD, D_H), jnp.float32))
_tc2 = pl.pallas_call(
    _tc2_body, out_shape=jax.ShapeDtypeStruct((NPAD, D_H), jnp.float32))
_tc3 = pl.pallas_call(
    _tc3_body, out_shape=jax.ShapeDtypeStruct((1, D_OUT), jnp.float32))


def kernel(x, edge_index, W1, b1, W2, b2, W3, b3):
    src = edge_index[0]
    dst = edge_index[1]
    srcr = src.reshape(NW, NB, CPB, B)
    dstr = dst.reshape(NW, NB, CPB, B)
    zrows = jnp.zeros((RPT, D_H), jnp.float32)
    x_pad = jnp.pad(x, ((0, NPAD - N), (0, 0)))

    degs = _deg_kernel(srcr, dstr)
    pad0 = jnp.zeros((NPAD - N,), jnp.float32)
    norm_src = jnp.concatenate(
        [lax.rsqrt(degs[0, 0, :N] + degs[1, 0, :N] + 1.0), pad0])
    norm_dst = jnp.concatenate(
        [lax.rsqrt(degs[0, 1, :N] + degs[1, 1, :N] + 1.0), pad0])

    g1 = _tc1(x_pad, norm_src[:, None], W1)
    m1a, m1b, cparts = _prop_c_kernel(g1, srcr, dstr, zrows, norm_dst)

    g2 = _tc2(m1a, m1b, norm_dst[:, None], norm_src[:, None],
              b1.reshape(1, D_H), W2)
    m2a, m2b = _prop_kernel(g2, srcr, dstr, zrows)

    c = cparts[0] + cparts[1] + norm_dst
    wvec = (c * norm_src * (1.0 / N))[:, None]
    out = _tc3(m2a, m2b, norm_dst[:, None], b2.reshape(1, D_H), wvec,
               W3, b3.reshape(1, D_OUT))
    return out.reshape(D_OUT)


# speedup vs baseline: 23.4565x; 1.1613x over previous
import jax
import jax.numpy as jnp
from jax import lax
from jax.experimental import pallas as pl
from jax.experimental.pallas import tpu as pltpu
from jax.experimental.pallas import tpu_sc as plsc

N = 10000
E = 320000
D_IN = 128
D_H = 128
D_OUT = 64

NC = 2
NS = 16
NW = NC * NS
EPW = E // NW
B = 80
NCH = EPW // B
NB = 5
CPB = NCH // NB
NPAD = 10240
SPT = NPAD // NS
RPT = NPAD // NS

_mesh = plsc.VectorSubcoreMesh(core_axis_name="c", subcore_axis_name="s")


def _fill(ref, nelem, value):
    v16 = jnp.full((16,), value, jnp.float32)

    @pl.loop(0, nelem // 16)
    def _(i):
        ref[pl.ds(i * 16, 16)] = v16


def _deg_body(srcr, dstr, degs_out, dego_sh, degi_sh, stag_s, stag_d,
              ones_v, zero_v, semo, semi):
    cid = lax.axis_index("c")
    sid = lax.axis_index("s")
    w = cid * NS + sid
    base = sid * SPT

    _fill(ones_v, B, 1.0)
    _fill(zero_v, SPT, 0.0)
    pltpu.sync_copy(zero_v, dego_sh.at[pl.ds(base, SPT)])
    pltpu.sync_copy(zero_v, degi_sh.at[pl.ds(base, SPT)])
    plsc.subcore_barrier()

    pltpu.sync_copy(srcr.at[w], stag_s)
    pltpu.sync_copy(dstr.at[w], stag_d)

    def wait_pair():
        pltpu.make_async_copy(ones_v, dego_sh.at[stag_s.at[0, 0]],
                              semo).wait()
        pltpu.make_async_copy(ones_v, degi_sh.at[stag_d.at[0, 0]],
                              semi).wait()

    @pl.loop(0, NB)
    def _(b):
        @pl.loop(0, CPB)
        def _(j):
            pltpu.async_copy(ones_v, dego_sh.at[stag_s.at[b, j]], semo,
                             add=True)
            pltpu.async_copy(ones_v, degi_sh.at[stag_d.at[b, j]], semi,
                             add=True)

            @pl.when(b * CPB + j >= 4)
            def _():
                wait_pair()

    for _ in range(4):
        wait_pair()

    plsc.subcore_barrier()
    pltpu.sync_copy(dego_sh.at[pl.ds(base, SPT)],
                    degs_out.at[cid, 0, pl.ds(base, SPT)])
    pltpu.sync_copy(degi_sh.at[pl.ds(base, SPT)],
                    degs_out.at[cid, 1, pl.ds(base, SPT)])


_deg_kernel = pl.kernel(
    _deg_body,
    out_type=jax.ShapeDtypeStruct((NC, 2, NPAD), jnp.float32),
    mesh=_mesh,
    scratch_types=[
        pltpu.VMEM_SHARED((NPAD,), jnp.float32),
        pltpu.VMEM_SHARED((NPAD,), jnp.float32),
        pltpu.VMEM((NB, CPB, B), jnp.int32),
        pltpu.VMEM((NB, CPB, B), jnp.int32),
        pltpu.VMEM((B,), jnp.float32),
        pltpu.VMEM((SPT,), jnp.float32),
        pltpu.SemaphoreType.DMA,
        pltpu.SemaphoreType.DMA,
    ],
)


def _make_prop_body(with_c):
    def body(g, srcr, dstr, zrows, *rest):
        if with_c:
            (normd, ma, mb, c_out, acc_sh, c_sh, stag_s, stag_d,
             rb0, rb1, rb2, vb0, vb1, vb2,
             rg0, rg1, rg2, rs0, rs1, rs2,
             cg0, cg1, cg2, cs0, cs1, cs2) = rest
            RB = ((rb0, rg0, rs0), (rb1, rg1, rs1), (rb2, rg2, rs2))
            VB = ((vb0, cg0, cs0), (vb1, cg1, cs1), (vb2, cg2, cs2))
        else:
            (ma, mb, acc_sh, stag_s, stag_d, rb0, rb1, rb2,
             rg0, rg1, rg2, rs0, rs1, rs2) = rest
            RB = ((rb0, rg0, rs0), (rb1, rg1, rs1), (rb2, rg2, rs2))

        cid = lax.axis_index("c")
        sid = lax.axis_index("s")
        w = cid * NS + sid
        rbase = sid * RPT
        base = sid * SPT

        if with_c:
            _fill(vb0, B, 0.0)
            for k in range(SPT // B):
                pltpu.sync_copy(vb0, c_sh.at[pl.ds(base + k * B, B)])

        @pl.when(cid == 0)
        def _():
            pltpu.sync_copy(g.at[pl.ds(rbase, RPT)],
                            acc_sh.at[pl.ds(rbase, RPT)])

        @pl.when(cid == 1)
        def _():
            pltpu.sync_copy(zrows, acc_sh.at[pl.ds(rbase, RPT)])

        plsc.subcore_barrier()

        def rgf(j, bi):
            pltpu.async_copy(g.at[stag_s.at[j]], RB[bi][0], RB[bi][1])

        def rgw(bi):
            pltpu.make_async_copy(g.at[stag_s.at[0]], RB[bi][0],
                                  RB[bi][1]).wait()

        def rsf(j, bi):
            pltpu.async_copy(RB[bi][0], acc_sh.at[stag_d.at[j]], RB[bi][2],
                             add=True)

        def rsw(bi):
            pltpu.make_async_copy(RB[bi][0], acc_sh.at[stag_d.at[0]],
                                  RB[bi][2]).wait()

        if with_c:
            def cgf(j, bi):
                pltpu.async_copy(normd.at[stag_d.at[j]], VB[bi][0],
                                 VB[bi][1])

            def cgw(bi):
                pltpu.make_async_copy(normd.at[stag_d.at[0]], VB[bi][0],
                                      VB[bi][1]).wait()

            def csf(j, bi):
                pltpu.async_copy(VB[bi][0], c_sh.at[stag_s.at[j]],
                                 VB[bi][2], add=True)

            def csw(bi):
                pltpu.make_async_copy(VB[bi][0], c_sh.at[stag_s.at[0]],
                                      VB[bi][2]).wait()

        def fire(j, bi):
            rgf(j, bi)
            if with_c:
                cgf(j, bi)

        def phase(j, bi, jn=None, bn=None, first=False):
            rgw(bi)
            rsf(j, bi)
            if with_c:
                cgw(bi)
                csf(j, bi)
            if jn is not None:
                if not first:
                    rsw(bn)
                    if with_c:
                        csw(bn)
                fire(jn, bn)

        @pl.loop(0, NB)
        def _(b):
            pltpu.sync_copy(srcr.at[w, b], stag_s)
            pltpu.sync_copy(dstr.at[w, b], stag_d)
            fire(0, 0)
            fire(1, 1)
            phase(0, 0, 2, 2, first=True)
            phase(1, 1, 3, 0)

            @pl.loop(0, (CPB - 4) // 3)
            def _(t):
                j = 2 + 3 * t
                phase(j, 2, j + 2, 1)
                phase(j + 1, 0, j + 3, 2)
                phase(j + 2, 1, j + 4, 0)

            phase(CPB - 2, 2)
            phase(CPB - 1, 0)
            for bi in range(3):
                rsw(bi)
                if with_c:
                    csw(bi)

        plsc.subcore_barrier()

        @pl.when(cid == 0)
        def _():
            pltpu.sync_copy(acc_sh.at[pl.ds(rbase, RPT)],
                            ma.at[pl.ds(rbase, RPT)])

        @pl.when(cid == 1)
        def _():
            pltpu.sync_copy(acc_sh.at[pl.ds(rbase, RPT)],
                            mb.at[pl.ds(rbase, RPT)])

        if with_c:
            pltpu.sync_copy(c_sh.at[pl.ds(base, SPT)],
                            c_out.at[cid, pl.ds(base, SPT)])

    return body


_prop_scratch = [
    pltpu.VMEM_SHARED((NPAD, D_H), jnp.float32),
    pltpu.VMEM((CPB, B), jnp.int32),
    pltpu.VMEM((CPB, B), jnp.int32),
    pltpu.VMEM((B, D_H), jnp.float32),
    pltpu.VMEM((B, D_H), jnp.float32),
    pltpu.VMEM((B, D_H), jnp.float32),
] + [pltpu.SemaphoreType.DMA] * 6

_prop_c_scratch = [
    pltpu.VMEM_SHARED((NPAD, D_H), jnp.float32),
    pltpu.VMEM_SHARED((NPAD,), jnp.float32),
    pltpu.VMEM((CPB, B), jnp.int32),
    pltpu.VMEM((CPB, B), jnp.int32),
    pltpu.VMEM((B, D_H), jnp.float32),
    pltpu.VMEM((B, D_H), jnp.float32),
    pltpu.VMEM((B, D_H), jnp.float32),
    pltpu.VMEM((B,), jnp.float32),
    pltpu.VMEM((B,), jnp.float32),
    pltpu.VMEM((B,), jnp.float32),
] + [pltpu.SemaphoreType.DMA] * 12

_prop_kernel = pl.kernel(
    _make_prop_body(with_c=False),
    out_type=(
        jax.ShapeDtypeStruct((NPAD, D_H), jnp.float32),
        jax.ShapeDtypeStruct((NPAD, D_H), jnp.float32),
    ),
    mesh=_mesh,
    scratch_types=_prop_scratch,
)

_prop_c_kernel = pl.kernel(
    _make_prop_body(with_c=True),
    out_type=(
        jax.ShapeDtypeStruct((NPAD, D_H), jnp.float32),
        jax.ShapeDtypeStruct((NPAD, D_H), jnp.float32),
        jax.ShapeDtypeStruct((NC, NPAD), jnp.float32),
    ),
    mesh=_mesh,
    scratch_types=_prop_c_scratch,
)


def _tc1_body(x_ref, ns_ref, w_ref, o_ref):
    o_ref[...] = jnp.dot(x_ref[...] * ns_ref[...], w_ref[...],
                         preferred_element_type=jnp.float32)


def _tc2_body(ma_ref, mb_ref, nd_ref, ns_ref, b_ref, w_ref, o_ref):
    h = jnp.maximum((ma_ref[...] + mb_ref[...]) * nd_ref[...] + b_ref[...],
                    0.0)
    o_ref[...] = jnp.dot(h * ns_ref[...], w_ref[...],
                         preferred_element_type=jnp.float32)


def _tc3_body(ma_ref, mb_ref, nd_ref, b2_ref, wv_ref, w3_ref, b3_ref, o_ref):
    h = jnp.maximum((ma_ref[...] + mb_ref[...]) * nd_ref[...] + b2_ref[...],
                    0.0)
    u = jnp.sum(h * wv_ref[...], axis=0, keepdims=True)
    o_ref[...] = jnp.dot(u, w3_ref[...],
                         preferred_element_type=jnp.float32) + b3_ref[...]


_tc1 = pl.pallas_call(
    _tc1_body, out_shape=jax.ShapeDtypeStruct((NPAD, D_H), jnp.float32))
_tc2 = pl.pallas_call(
    _tc2_body, out_shape=jax.ShapeDtypeStruct((NPAD, D_H), jnp.float32))
_tc3 = pl.pallas_call(
    _tc3_body, out_shape=jax.ShapeDtypeStruct((1, D_OUT), jnp.float32))


def kernel(x, edge_index, W1, b1, W2, b2, W3, b3):
    src = edge_index[0]
    dst = edge_index[1]
    srcr = src.reshape(NW, NB, CPB, B)
    dstr = dst.reshape(NW, NB, CPB, B)
    zrows = jnp.zeros((RPT, D_H), jnp.float32)
    x_pad = jnp.pad(x, ((0, NPAD - N), (0, 0)))

    degs = _deg_kernel(srcr, dstr)
    pad0 = jnp.zeros((NPAD - N,), jnp.float32)
    norm_src = jnp.concatenate(
        [lax.rsqrt(degs[0, 0, :N] + degs[1, 0, :N] + 1.0), pad0])
    norm_dst = jnp.concatenate(
        [lax.rsqrt(degs[0, 1, :N] + degs[1, 1, :N] + 1.0), pad0])

    g1 = _tc1(x_pad, norm_src[:, None], W1)
    m1a, m1b, cparts = _prop_c_kernel(g1, srcr, dstr, zrows, norm_dst)

    g2 = _tc2(m1a, m1b, norm_dst[:, None], norm_src[:, None],
              b1.reshape(1, D_H), W2)
    m2a, m2b = _prop_kernel(g2, srcr, dstr, zrows)

    c = cparts[0] + cparts[1] + norm_dst
    wvec = (c * norm_src * (1.0 / N))[:, None]
    out = _tc3(m2a, m2b, norm_dst[:, None], b2.reshape(1, D_H), wvec,
               W3, b3.reshape(1, D_OUT))
    return out.reshape(D_OUT)


# speedup vs baseline: 23.7137x; 1.0110x over previous
import jax
import jax.numpy as jnp
from jax import lax
from jax.experimental import pallas as pl
from jax.experimental.pallas import tpu as pltpu
from jax.experimental.pallas import tpu_sc as plsc

N = 10000
E = 320000
D_IN = 128
D_H = 128
D_OUT = 64

NC = 2
NS = 16
NW = NC * NS
EPW = E // NW
B = 80
NCH = EPW // B
NB = 5
CPB = NCH // NB
NPAD = 10240
SPT = NPAD // NS
RPT = NPAD // NS

_mesh = plsc.VectorSubcoreMesh(core_axis_name="c", subcore_axis_name="s")


def _fill(ref, nelem, value):
    v16 = jnp.full((16,), value, jnp.float32)

    @pl.loop(0, nelem // 16)
    def _(i):
        ref[pl.ds(i * 16, 16)] = v16


def _deg_body(srcr, dstr, degs_out, dego_sh, degi_sh, stag_s, stag_d,
              ones_v, zero_v, semo, semi):
    cid = lax.axis_index("c")
    sid = lax.axis_index("s")
    w = cid * NS + sid
    base = sid * SPT

    _fill(ones_v, B, 1.0)
    _fill(zero_v, SPT, 0.0)
    pltpu.sync_copy(zero_v, dego_sh.at[pl.ds(base, SPT)])
    pltpu.sync_copy(zero_v, degi_sh.at[pl.ds(base, SPT)])
    plsc.subcore_barrier()

    pltpu.sync_copy(srcr.at[w], stag_s)
    pltpu.sync_copy(dstr.at[w], stag_d)

    def wait_pair():
        pltpu.make_async_copy(ones_v, dego_sh.at[stag_s.at[0, 0]],
                              semo).wait()
        pltpu.make_async_copy(ones_v, degi_sh.at[stag_d.at[0, 0]],
                              semi).wait()

    @pl.loop(0, NB)
    def _(b):
        @pl.loop(0, CPB)
        def _(j):
            pltpu.async_copy(ones_v, dego_sh.at[stag_s.at[b, j]], semo,
                             add=True)
            pltpu.async_copy(ones_v, degi_sh.at[stag_d.at[b, j]], semi,
                             add=True)

            @pl.when(b * CPB + j >= 4)
            def _():
                wait_pair()

    for _ in range(4):
        wait_pair()

    plsc.subcore_barrier()
    pltpu.sync_copy(dego_sh.at[pl.ds(base, SPT)],
                    degs_out.at[cid, 0, pl.ds(base, SPT)])
    pltpu.sync_copy(degi_sh.at[pl.ds(base, SPT)],
                    degs_out.at[cid, 1, pl.ds(base, SPT)])


_deg_kernel = pl.kernel(
    _deg_body,
    out_type=jax.ShapeDtypeStruct((NC, 2, NPAD), jnp.float32),
    mesh=_mesh,
    scratch_types=[
        pltpu.VMEM_SHARED((NPAD,), jnp.float32),
        pltpu.VMEM_SHARED((NPAD,), jnp.float32),
        pltpu.VMEM((NB, CPB, B), jnp.int32),
        pltpu.VMEM((NB, CPB, B), jnp.int32),
        pltpu.VMEM((B,), jnp.float32),
        pltpu.VMEM((SPT,), jnp.float32),
        pltpu.SemaphoreType.DMA,
        pltpu.SemaphoreType.DMA,
    ],
)


def _make_prop_body(with_c):
    def body(g, srcr, dstr, zrows, *rest):
        if with_c:
            (normd, ma, mb, c_out, acc_sh, c_sh, stag_s, stag_d,
             rb0, rb1, rb2, vb0, vb1, vb2,
             rg0, rg1, rg2, rs0, rs1, rs2,
             cg0, cg1, cg2, cs0, cs1, cs2) = rest
            RB = ((rb0, rg0, rs0), (rb1, rg1, rs1), (rb2, rg2, rs2))
            VB = ((vb0, cg0, cs0), (vb1, cg1, cs1), (vb2, cg2, cs2))
        else:
            (ma, mb, acc_sh, stag_s, stag_d, rb0, rb1, rb2,
             rg0, rg1, rg2, rs0, rs1, rs2) = rest
            RB = ((rb0, rg0, rs0), (rb1, rg1, rs1), (rb2, rg2, rs2))

        cid = lax.axis_index("c")
        sid = lax.axis_index("s")
        w = cid * NS + sid
        rbase = sid * RPT
        base = sid * SPT

        if with_c:
            _fill(vb0, B, 0.0)
            for k in range(SPT // B):
                pltpu.sync_copy(vb0, c_sh.at[pl.ds(base + k * B, B)])

        @pl.when(cid == 0)
        def _():
            pltpu.sync_copy(g.at[pl.ds(rbase, RPT)],
                            acc_sh.at[pl.ds(rbase, RPT)])

        @pl.when(cid == 1)
        def _():
            pltpu.sync_copy(zrows, acc_sh.at[pl.ds(rbase, RPT)])

        plsc.subcore_barrier()

        def rgf(j, bi):
            pltpu.async_copy(g.at[stag_s.at[j]], RB[bi][0], RB[bi][1])

        def rgw(bi):
            pltpu.make_async_copy(g.at[stag_s.at[0]], RB[bi][0],
                                  RB[bi][1]).wait()

        def rsf(j, bi):
            pltpu.async_copy(RB[bi][0], acc_sh.at[stag_d.at[j]], RB[bi][2],
                             add=True)

        def rsw(bi):
            pltpu.make_async_copy(RB[bi][0], acc_sh.at[stag_d.at[0]],
                                  RB[bi][2]).wait()

        if with_c:
            def cgf(j, bi):
                pltpu.async_copy(normd.at[stag_d.at[j]], VB[bi][0],
                                 VB[bi][1])

            def cgw(bi):
                pltpu.make_async_copy(normd.at[stag_d.at[0]], VB[bi][0],
                                      VB[bi][1]).wait()

            def csf(j, bi):
                pltpu.async_copy(VB[bi][0], c_sh.at[stag_s.at[j]],
                                 VB[bi][2], add=True)

            def csw(bi):
                pltpu.make_async_copy(VB[bi][0], c_sh.at[stag_s.at[0]],
                                      VB[bi][2]).wait()

        def fire(j, bi):
            rgf(j, bi)
            if with_c:
                cgf(j, bi)

        def phase(j, bi, jn=None, bn=None, first=False):
            rgw(bi)
            rsf(j, bi)
            if with_c:
                cgw(bi)
                csf(j, bi)
            if jn is not None:
                if not first:
                    rsw(bn)
                    if with_c:
                        csw(bn)
                fire(jn, bn)

        @pl.loop(0, NB)
        def _(b):
            pltpu.sync_copy(srcr.at[w, b], stag_s)
            pltpu.sync_copy(dstr.at[w, b], stag_d)
            fire(0, 0)
            fire(1, 1)
            phase(0, 0, 2, 2, first=True)
            phase(1, 1, 3, 0)

            @pl.loop(0, (CPB - 4) // 3)
            def _(t):
                j = 2 + 3 * t
                phase(j, 2, j + 2, 1)
                phase(j + 1, 0, j + 3, 2)
                phase(j + 2, 1, j + 4, 0)

            phase(CPB - 2, 2)
            phase(CPB - 1, 0)
            for bi in range(3):
                rsw(bi)
                if with_c:
                    csw(bi)

        plsc.subcore_barrier()

        @pl.when(cid == 0)
        def _():
            pltpu.sync_copy(acc_sh.at[pl.ds(rbase, RPT)],
                            ma.at[pl.ds(rbase, RPT)])

        @pl.when(cid == 1)
        def _():
            pltpu.sync_copy(acc_sh.at[pl.ds(rbase, RPT)],
                            mb.at[pl.ds(rbase, RPT)])

        if with_c:
            pltpu.sync_copy(c_sh.at[pl.ds(base, SPT)],
                            c_out.at[cid, pl.ds(base, SPT)])

    return body


_prop_scratch = [
    pltpu.VMEM_SHARED((NPAD, D_H), jnp.float32),
    pltpu.VMEM((CPB, B), jnp.int32),
    pltpu.VMEM((CPB, B), jnp.int32),
    pltpu.VMEM((B, D_H), jnp.float32),
    pltpu.VMEM((B, D_H), jnp.float32),
    pltpu.VMEM((B, D_H), jnp.float32),
] + [pltpu.SemaphoreType.DMA] * 6

_prop_c_scratch = [
    pltpu.VMEM_SHARED((NPAD, D_H), jnp.float32),
    pltpu.VMEM_SHARED((NPAD,), jnp.float32),
    pltpu.VMEM((CPB, B), jnp.int32),
    pltpu.VMEM((CPB, B), jnp.int32),
    pltpu.VMEM((B, D_H), jnp.float32),
    pltpu.VMEM((B, D_H), jnp.float32),
    pltpu.VMEM((B, D_H), jnp.float32),
    pltpu.VMEM((B,), jnp.float32),
    pltpu.VMEM((B,), jnp.float32),
    pltpu.VMEM((B,), jnp.float32),
] + [pltpu.SemaphoreType.DMA] * 12

_prop_kernel = pl.kernel(
    _make_prop_body(with_c=False),
    out_type=(
        jax.ShapeDtypeStruct((NPAD, D_H), jnp.float32),
        jax.ShapeDtypeStruct((NPAD, D_H), jnp.float32),
    ),
    mesh=_mesh,
    scratch_types=_prop_scratch,
)

_prop_c_kernel = pl.kernel(
    _make_prop_body(with_c=True),
    out_type=(
        jax.ShapeDtypeStruct((NPAD, D_H), jnp.float32),
        jax.ShapeDtypeStruct((NPAD, D_H), jnp.float32),
        jax.ShapeDtypeStruct((NC, NPAD), jnp.float32),
    ),
    mesh=_mesh,
    scratch_types=_prop_c_scratch,
)


def _tca_body(ma_ref, mb_ref, w_ref, nd_ref, ns_ref, b_ref, o_ref):
    m = jnp.dot(ma_ref[...] + mb_ref[...], w_ref[...],
                preferred_element_type=jnp.float32)
    o_ref[...] = jnp.maximum(m * nd_ref[...] + b_ref[...], 0.0) * ns_ref[...]


def _tcb_body(ma_ref, mb_ref, w_ref, nd_ref, b2_ref, wv_ref, w3_ref, b3_ref,
              o_ref):
    m = jnp.dot(ma_ref[...] + mb_ref[...], w_ref[...],
                preferred_element_type=jnp.float32)
    h = jnp.maximum(m * nd_ref[...] + b2_ref[...], 0.0)
    u = jnp.sum(h * wv_ref[...], axis=0, keepdims=True)
    o_ref[...] = jnp.dot(u, w3_ref[...],
                         preferred_element_type=jnp.float32) + b3_ref[...]


_tca = pl.pallas_call(
    _tca_body, out_shape=jax.ShapeDtypeStruct((NPAD, D_H), jnp.float32))
_tcb = pl.pallas_call(
    _tcb_body, out_shape=jax.ShapeDtypeStruct((1, D_OUT), jnp.float32))


def kernel(x, edge_index, W1, b1, W2, b2, W3, b3):
    src = edge_index[0]
    dst = edge_index[1]
    srcr = src.reshape(NW, NB, CPB, B)
    dstr = dst.reshape(NW, NB, CPB, B)
    zrows = jnp.zeros((RPT, D_H), jnp.float32)
    x_pad = jnp.pad(x, ((0, NPAD - N), (0, 0)))

    degs = _deg_kernel(srcr, dstr)
    pad0 = jnp.zeros((NPAD - N,), jnp.float32)
    norm_src = jnp.concatenate(
        [lax.rsqrt(degs[0, 0, :N] + degs[1, 0, :N] + 1.0), pad0])
    norm_dst = jnp.concatenate(
        [lax.rsqrt(degs[0, 1, :N] + degs[1, 1, :N] + 1.0), pad0])

    xs = x_pad * norm_src[:, None]
    m1a, m1b, cparts = _prop_c_kernel(xs, srcr, dstr, zrows, norm_dst)

    h1s = _tca(m1a, m1b, W1, norm_dst[:, None], norm_src[:, None],
               b1.reshape(1, D_H))
    m2a, m2b = _prop_kernel(h1s, srcr, dstr, zrows)

    c = cparts[0] + cparts[1] + norm_dst
    wvec = (c * norm_src * (1.0 / N))[:, None]
    out = _tcb(m2a, m2b, W2, norm_dst[:, None], b2.reshape(1, D_H), wvec,
               W3, b3.reshape(1, D_OUT))
    return out.reshape(D_OUT)
